# Initial kernel scaffold; baseline (speedup 1.0000x reference)
import jax
import jax.numpy as jnp
from jax.experimental import pallas as pl


def kernel(user, mission, user_table, mission_table, bias):
    raise NotImplementedError("write your pallas kernel here")


# speedup vs baseline: 1.0817x; 1.0817x over previous
import functools

import jax
import jax.numpy as jnp
from jax import lax
from jax.experimental import pallas as pl
from jax.experimental.pallas import tpu as pltpu
from jax.experimental.pallas import tpu_sc as plsc

BATCH = 16384
NC = 2
NS = 16
NW = NC * NS
LANES = 16
BPW = BATCH // NW
CHUNK = 128
NCHUNK = BPW // CHUNK


def _sc_lookup_sum(user_idx, mission_idx, user_vec, mission_vec, bias16):
    mesh = plsc.VectorSubcoreMesh(core_axis_name="c", subcore_axis_name="s")

    @functools.partial(
        pl.kernel,
        mesh=mesh,
        out_type=jax.ShapeDtypeStruct((NW, NCHUNK, CHUNK), jnp.float32),
        scratch_types=[
            pltpu.VMEM((NCHUNK, CHUNK), jnp.int32),
            pltpu.VMEM((NCHUNK, CHUNK), jnp.int32),
            pltpu.VMEM((NCHUNK, CHUNK), jnp.float32),
            pltpu.VMEM((NCHUNK, CHUNK), jnp.float32),
            pltpu.VMEM((LANES,), jnp.float32),
            pltpu.SemaphoreType.DMA,
        ],
    )
    def k(uidx_hbm, midx_hbm, ut_hbm, mt_hbm, bias_hbm, out_hbm,
          uix, mix, uval, mval, bv, sem):
        wid = lax.axis_index("s") * NC + lax.axis_index("c")
        pltpu.sync_copy(bias_hbm, bv)
        pltpu.sync_copy(uidx_hbm.at[wid], uix)
        pltpu.sync_copy(midx_hbm.at[wid], mix)
        copies = []
        for j in range(NCHUNK):
            copies.append(pltpu.async_copy(ut_hbm.at[uix.at[j]], uval.at[j], sem))
            copies.append(pltpu.async_copy(mt_hbm.at[mix.at[j]], mval.at[j], sem))
        for c in copies:
            c.wait()
        b = bv[...]
        for j in range(NCHUNK):
            for t in range(CHUNK // LANES):
                s = pl.ds(t * LANES, LANES)
                uval[j, s] = uval[j, s] + mval[j, s] + b
        pltpu.sync_copy(uval, out_hbm.at[wid])

    return k(user_idx, mission_idx, user_vec, mission_vec, bias16)


def kernel(user, mission, user_table, mission_table, bias):
    uidx = user.astype(jnp.int32).reshape(NW, NCHUNK, CHUNK)
    midx = mission.astype(jnp.int32).reshape(NW, NCHUNK, CHUNK)
    ut = user_table.reshape(-1)
    mt = mission_table.reshape(-1)
    bias16 = jnp.broadcast_to(bias.reshape(()), (LANES,)).astype(jnp.float32)
    out = _sc_lookup_sum(uidx, midx, ut, mt, bias16)
    return out.reshape(BATCH)


# speedup vs baseline: 1.0921x; 1.0096x over previous
import functools

import jax
import jax.numpy as jnp
from jax import lax
from jax.experimental import pallas as pl
from jax.experimental.pallas import tpu as pltpu
from jax.experimental.pallas import tpu_sc as plsc

BATCH = 16384
NC = 2
NS = 16
NW = NC * NS
LANES = 16
BPW = BATCH // NW
CHUNK = 128
NCHUNK = BPW // CHUNK


def _sc_lookup_sum(idx_pairs, user_vec, mission_vec, bias16):
    mesh = plsc.VectorSubcoreMesh(core_axis_name="c", subcore_axis_name="s")

    @functools.partial(
        pl.kernel,
        mesh=mesh,
        out_type=jax.ShapeDtypeStruct((NW, NCHUNK, CHUNK), jnp.float32),
        scratch_types=[
            pltpu.VMEM((2 * NCHUNK, CHUNK), jnp.int32),
            pltpu.VMEM((NCHUNK, CHUNK), jnp.float32),
            pltpu.VMEM((NCHUNK, CHUNK), jnp.float32),
            pltpu.VMEM((LANES,), jnp.float32),
            pltpu.SemaphoreType.DMA,
            pltpu.SemaphoreType.DMA,
        ],
    )
    def k(idx_hbm, ut_hbm, mt_hbm, bias_hbm, out_hbm,
          ix, uval, mval, bv, sem, bsem):
        wid = lax.axis_index("s") * NC + lax.axis_index("c")
        bcp = pltpu.async_copy(bias_hbm, bv, bsem)
        pltpu.sync_copy(idx_hbm.at[wid], ix)
        copies = []
        for j in range(NCHUNK):
            copies.append(pltpu.async_copy(ut_hbm.at[ix.at[j]], uval.at[j], sem))
            copies.append(pltpu.async_copy(
                mt_hbm.at[ix.at[NCHUNK + j]], mval.at[j], sem))
        for c in copies:
            c.wait()
        bcp.wait()
        b = bv[...]
        for j in range(NCHUNK):
            for t in range(CHUNK // LANES):
                s = pl.ds(t * LANES, LANES)
                uval[j, s] = uval[j, s] + mval[j, s] + b
        pltpu.sync_copy(uval, out_hbm.at[wid])

    return k(idx_pairs, user_vec, mission_vec, bias16)


def kernel(user, mission, user_table, mission_table, bias):
    uidx = user.astype(jnp.int32).reshape(NW, NCHUNK, CHUNK)
    midx = mission.astype(jnp.int32).reshape(NW, NCHUNK, CHUNK)
    idx_pairs = jnp.concatenate([uidx, midx], axis=1)
    ut = user_table.reshape(-1)
    mt = mission_table.reshape(-1)
    bias16 = jnp.broadcast_to(bias.reshape(()), (LANES,)).astype(jnp.float32)
    out = _sc_lookup_sum(idx_pairs, ut, mt, bias16)
    return out.reshape(BATCH)


# speedup vs baseline: 1.0944x; 1.0021x over previous
import functools

import jax
import jax.numpy as jnp
from jax import lax
from jax.experimental import pallas as pl
from jax.experimental.pallas import tpu as pltpu
from jax.experimental.pallas import tpu_sc as plsc

BATCH = 16384
NC = 2
NS = 16
NW = NC * NS
LANES = 16
BPW = BATCH // NW
CHUNK = 128
NCHUNK = BPW // CHUNK


def _sc_lookup_sum(user_idx, mission_idx, user_vec, mission_vec, bias):
    mesh = plsc.VectorSubcoreMesh(core_axis_name="c", subcore_axis_name="s")

    @functools.partial(
        pl.kernel,
        mesh=mesh,
        out_type=jax.ShapeDtypeStruct((NW, NCHUNK, CHUNK), jnp.float32),
        scratch_types=[
            pltpu.VMEM((NCHUNK, CHUNK), jnp.int32),
            pltpu.VMEM((NCHUNK, CHUNK), jnp.int32),
            pltpu.VMEM((NCHUNK, CHUNK), jnp.float32),
            pltpu.VMEM((NCHUNK, CHUNK), jnp.float32),
            pltpu.VMEM((LANES,), jnp.float32),
            pltpu.SemaphoreType.DMA,
            pltpu.SemaphoreType.DMA,
            pltpu.SemaphoreType.DMA,
        ],
    )
    def k(uidx_hbm, midx_hbm, ut_hbm, mt_hbm, bias_hbm, out_hbm,
          uix, mix, uval, mval, bv, gsem, isem, wsem):
        wid = lax.axis_index("s") * NC + lax.axis_index("c")
        bcp = pltpu.async_copy(bias_hbm, bv, isem)
        ucp = pltpu.async_copy(uidx_hbm.at[wid], uix, isem)
        mcp = pltpu.async_copy(midx_hbm.at[wid], mix, isem)
        ucp.wait()
        mcp.wait()
        gathers = []
        for j in range(NCHUNK):
            gathers.append(
                (pltpu.async_copy(ut_hbm.at[uix.at[j]], uval.at[j], gsem),
                 pltpu.async_copy(mt_hbm.at[mix.at[j]], mval.at[j], gsem)))
        bcp.wait()
        b = bv[...]
        writes = []
        for j in range(NCHUNK):
            gu, gm = gathers[j]
            gu.wait()
            gm.wait()
            for t in range(CHUNK // LANES):
                s = pl.ds(t * LANES, LANES)
                uval[j, s] = uval[j, s] + mval[j, s] + b
            writes.append(
                pltpu.async_copy(uval.at[j], out_hbm.at[wid, j], wsem))
        for w in writes:
            w.wait()

    return k(user_idx, mission_idx, user_vec, mission_vec, bias)


def kernel(user, mission, user_table, mission_table, bias):
    uidx = user.astype(jnp.int32).reshape(NW, NCHUNK, CHUNK)
    midx = mission.astype(jnp.int32).reshape(NW, NCHUNK, CHUNK)
    ut = user_table.reshape(-1)
    mt = mission_table.reshape(-1)
    bias16 = jnp.broadcast_to(bias.reshape(()), (LANES,)).astype(jnp.float32)
    out = _sc_lookup_sum(uidx, midx, ut, mt, bias16)
    return out.reshape(BATCH)


# speedup vs baseline: 3.3070x; 3.0219x over previous
import functools

import jax
import jax.numpy as jnp
from jax import lax
from jax.experimental import pallas as pl
from jax.experimental.pallas import tpu as pltpu
from jax.experimental.pallas import tpu_sc as plsc

BATCH = 16384
NC = 2
NS = 16
NW = NC * NS
LANES = 16
BPW = BATCH // NW
CHUNK = 128
NCHUNK = BPW // CHUNK


def _sc_lookup_sum(user_idx, mission_idx, user_table, mission_table, bias16):
    mesh = plsc.VectorSubcoreMesh(core_axis_name="c", subcore_axis_name="s")

    @functools.partial(
        pl.kernel,
        mesh=mesh,
        out_type=jax.ShapeDtypeStruct((BATCH,), jnp.float32),
        scratch_types=[
            pltpu.VMEM((NCHUNK, CHUNK), jnp.int32),
            pltpu.VMEM((NCHUNK, CHUNK), jnp.int32),
            pltpu.VMEM((NCHUNK, CHUNK), jnp.float32),
            pltpu.VMEM((NCHUNK, CHUNK), jnp.float32),
            pltpu.VMEM((LANES,), jnp.float32),
            pltpu.SemaphoreType.DMA,
            pltpu.SemaphoreType.DMA,
            pltpu.SemaphoreType.DMA,
        ],
    )
    def k(uidx_hbm, midx_hbm, ut_hbm, mt_hbm, bias_hbm, out_hbm,
          uix, mix, uval, mval, bv, gsem, isem, wsem):
        wid = lax.axis_index("s") * NC + lax.axis_index("c")
        base = wid * BPW
        bcp = pltpu.async_copy(bias_hbm, bv, isem)
        idx_copies = []
        for j in range(NCHUNK):
            src = pl.ds(base + j * CHUNK, CHUNK)
            idx_copies.append(
                pltpu.async_copy(uidx_hbm.at[src], uix.at[j], isem))
            idx_copies.append(
                pltpu.async_copy(midx_hbm.at[src], mix.at[j], isem))
        gathers = []
        for j in range(NCHUNK):
            idx_copies[2 * j].wait()
            idx_copies[2 * j + 1].wait()
            gathers.append(
                (pltpu.async_copy(ut_hbm.at[0].at[uix.at[j]], uval.at[j], gsem),
                 pltpu.async_copy(mt_hbm.at[0].at[mix.at[j]], mval.at[j], gsem)))
        bcp.wait()
        b = bv[...]
        writes = []
        for j in range(NCHUNK):
            gu, gm = gathers[j]
            gu.wait()
            gm.wait()
            for t in range(CHUNK // LANES):
                s = pl.ds(t * LANES, LANES)
                uval[j, s] = uval[j, s] + mval[j, s] + b
            writes.append(pltpu.async_copy(
                uval.at[j], out_hbm.at[pl.ds(base + j * CHUNK, CHUNK)], wsem))
        for w in writes:
            w.wait()

    return k(user_idx, mission_idx, user_table, mission_table, bias16)


def kernel(user, mission, user_table, mission_table, bias):
    bias16 = jnp.broadcast_to(bias.reshape(()), (LANES,)).astype(jnp.float32)
    out = _sc_lookup_sum(user.astype(jnp.int32), mission.astype(jnp.int32),
                         user_table.reshape(1, -1), mission_table.reshape(1, -1),
                         bias16)
    return out
